# Initial kernel scaffold; baseline (speedup 1.0000x reference)
import jax
import jax.numpy as jnp
from jax.experimental import pallas as pl


def kernel(ids, weight_fp):
    raise NotImplementedError("write your pallas kernel here")


# speedup vs baseline: 5.1871x; 5.1871x over previous
import functools

import jax
import jax.numpy as jnp
from jax import lax
from jax.experimental import pallas as pl
from jax.experimental.pallas import tpu as pltpu
from jax.experimental.pallas import tpu_sc as plsc


@functools.lru_cache(maxsize=None)
def _make_sc_gather(B, V, D, dtype_name):
    dtype = jnp.dtype(dtype_name)
    info = plsc.get_sparse_core_info()
    NC, NS = info.num_cores, info.num_subcores
    NW = NC * NS
    assert B % NW == 0
    b_per_w = B // NW
    C = 128
    assert b_per_w % C == 0
    n_chunks = b_per_w // C
    mesh = plsc.VectorSubcoreMesh(core_axis_name="c", subcore_axis_name="s")

    @functools.partial(
        pl.kernel,
        mesh=mesh,
        out_type=jax.ShapeDtypeStruct((B, D), dtype),
        scratch_types=[
            pltpu.VMEM((C,), jnp.int32),
            pltpu.VMEM((C, D), dtype),
            pltpu.SemaphoreType.DMA,
        ],
    )
    def k(table_hbm, idx_hbm, out_hbm, idx_v, rows_v, sem):
        wid = lax.axis_index("s") * NC + lax.axis_index("c")
        base = wid * b_per_w

        def body(g, carry):
            off = base + g * C
            pltpu.sync_copy(idx_hbm.at[pl.ds(off, C)], idx_v)
            pltpu.async_copy(table_hbm.at[idx_v], rows_v, sem).wait()
            pltpu.sync_copy(rows_v, out_hbm.at[pl.ds(off, C)])
            return carry

        lax.fori_loop(0, n_chunks, body, 0)

    return k


def kernel(ids, weight_fp):
    V, D = weight_fp.shape
    ids_flat = ids.reshape(-1).astype(jnp.int32)
    B = ids_flat.shape[0]
    out = _make_sc_gather(B, V, D, weight_fp.dtype.name)(weight_fp, ids_flat)
    return out.reshape(*ids.shape, D)


# speedup vs baseline: 9.2210x; 1.7777x over previous
import functools

import jax
import jax.numpy as jnp
from jax import lax
from jax.experimental import pallas as pl
from jax.experimental.pallas import tpu as pltpu
from jax.experimental.pallas import tpu_sc as plsc

_C = 128
_NBUF = 4
_LA = 2


@functools.lru_cache(maxsize=None)
def _make_sc_gather(B, V, D, dtype_name):
    dtype = jnp.dtype(dtype_name)
    info = plsc.get_sparse_core_info()
    NC, NS = info.num_cores, info.num_subcores
    NW = NC * NS
    assert B % NW == 0
    b_per_w = B // NW
    C, NBUF = _C, _NBUF
    assert b_per_w % C == 0
    n_chunks = b_per_w // C
    assert n_chunks % NBUF == 0 and n_chunks >= 2 * NBUF
    mesh = plsc.VectorSubcoreMesh(core_axis_name="c", subcore_axis_name="s")

    @functools.partial(
        pl.kernel,
        mesh=mesh,
        out_type=jax.ShapeDtypeStruct((B, D), dtype),
        scratch_types=[
            pltpu.VMEM((b_per_w,), jnp.int32),
            pltpu.VMEM((NBUF, C, D), dtype),
            pltpu.SemaphoreType.DMA((NBUF,)),
            pltpu.SemaphoreType.DMA((NBUF,)),
        ],
    )
    def k(table_hbm, idx_hbm, out_hbm, idx_v, rows, gsem, ssem):
        wid = lax.axis_index("s") * NC + lax.axis_index("c")
        base = wid * b_per_w
        pltpu.sync_copy(idx_hbm.at[pl.ds(base, b_per_w)], idx_v)

        def start_gather(g, b):
            pltpu.async_copy(
                table_hbm.at[idx_v.at[pl.ds(g * C, C)]], rows.at[b], gsem.at[b]
            )

        def wait_gather(b):
            pltpu.make_async_copy(
                table_hbm.at[idx_v.at[pl.ds(0, C)]], rows.at[b], gsem.at[b]
            ).wait()

        def start_store(g, b):
            pltpu.async_copy(
                rows.at[b], out_hbm.at[pl.ds(base + g * C, C)], ssem.at[b]
            )

        def wait_store(b):
            pltpu.make_async_copy(
                rows.at[b], out_hbm.at[pl.ds(base, C)], ssem.at[b]
            ).wait()

        start_gather(0, 0)
        start_gather(1, 1)
        start_gather(2, 2)
        wait_gather(0)
        start_store(0, 0)
        start_gather(3, 3)
        wait_gather(1)
        start_store(1, 1)

        def body(o, carry):
            g0 = o * NBUF
            for b in range(NBUF):
                g = g0 + b
                wait_store(b)
                start_gather(g, b)
                wait_gather((b + _LA) % NBUF)
                start_store(g - _LA, (b + _LA) % NBUF)
            return carry

        lax.fori_loop(1, n_chunks // NBUF, body, 0)

        wait_gather((n_chunks - 2) % NBUF)
        start_store(n_chunks - 2, (n_chunks - 2) % NBUF)
        wait_gather((n_chunks - 1) % NBUF)
        start_store(n_chunks - 1, (n_chunks - 1) % NBUF)
        for b in range(NBUF):
            wait_store(b)

    return k


def kernel(ids, weight_fp):
    V, D = weight_fp.shape
    ids_flat = ids.reshape(-1).astype(jnp.int32)
    B = ids_flat.shape[0]
    out = _make_sc_gather(B, V, D, weight_fp.dtype.name)(weight_fp, ids_flat)
    return out.reshape(*ids.shape, D)


# speedup vs baseline: 9.2424x; 1.0023x over previous
import functools

import jax
import jax.numpy as jnp
from jax import lax
from jax.experimental import pallas as pl
from jax.experimental.pallas import tpu as pltpu
from jax.experimental.pallas import tpu_sc as plsc

_C = 128
_NBUF = 5
_LA = 3


@functools.lru_cache(maxsize=None)
def _make_sc_gather(B, V, D, dtype_name):
    dtype = jnp.dtype(dtype_name)
    info = plsc.get_sparse_core_info()
    NC, NS = info.num_cores, info.num_subcores
    NW = NC * NS
    assert B % NW == 0
    b_per_w = B // NW
    C, NBUF = _C, _NBUF
    assert b_per_w % C == 0
    n_chunks = b_per_w // C
    assert n_chunks % NBUF == 0 and n_chunks >= 2 * NBUF
    mesh = plsc.VectorSubcoreMesh(core_axis_name="c", subcore_axis_name="s")

    @functools.partial(
        pl.kernel,
        mesh=mesh,
        out_type=jax.ShapeDtypeStruct((B, D), dtype),
        scratch_types=[
            pltpu.VMEM((b_per_w,), jnp.int32),
            pltpu.VMEM((NBUF, C, D), dtype),
            pltpu.SemaphoreType.DMA((NBUF,)),
            pltpu.SemaphoreType.DMA((NBUF,)),
        ],
    )
    def k(table_hbm, idx_hbm, out_hbm, idx_v, rows, gsem, ssem):
        wid = lax.axis_index("s") * NC + lax.axis_index("c")
        base = wid * b_per_w
        pltpu.sync_copy(idx_hbm.at[pl.ds(base, b_per_w)], idx_v)

        def start_gather(g, b):
            pltpu.async_copy(
                table_hbm.at[idx_v.at[pl.ds(g * C, C)]], rows.at[b], gsem.at[b]
            )

        def wait_gather(b):
            pltpu.make_async_copy(
                table_hbm.at[idx_v.at[pl.ds(0, C)]], rows.at[b], gsem.at[b]
            ).wait()

        def start_store(g, b):
            pltpu.async_copy(
                rows.at[b], out_hbm.at[pl.ds(base + g * C, C)], ssem.at[b]
            )

        def wait_store(b):
            pltpu.make_async_copy(
                rows.at[b], out_hbm.at[pl.ds(base, C)], ssem.at[b]
            ).wait()

        for g in range(NBUF):
            start_gather(g, g)
            if g >= _LA:
                wait_gather(g - _LA)
                start_store(g - _LA, g - _LA)

        def body(o, carry):
            g0 = o * NBUF
            for b in range(NBUF):
                g = g0 + b
                wait_store(b)
                start_gather(g, b)
                wait_gather((b - _LA) % NBUF)
                start_store(g - _LA, (b - _LA) % NBUF)
            return carry

        lax.fori_loop(1, n_chunks // NBUF, body, 0)

        for g in range(n_chunks - _LA, n_chunks):
            wait_gather(g % NBUF)
            start_store(g, g % NBUF)
        for b in range(NBUF):
            wait_store(b)

    return k


def kernel(ids, weight_fp):
    V, D = weight_fp.shape
    ids_flat = ids.reshape(-1).astype(jnp.int32)
    B = ids_flat.shape[0]
    out = _make_sc_gather(B, V, D, weight_fp.dtype.name)(weight_fp, ids_flat)
    return out.reshape(*ids.shape, D)
